# baseline (speedup 1.0000x reference)
import functools

import jax
import jax.numpy as jnp
from jax import lax
from jax.experimental import pallas as pl
from jax.experimental.pallas import tpu as pltpu
from jax.experimental.pallas import tpu_sc as plsc

_NUM_ENTS = 10000
_NUM_RELS2 = 400
_NUM_PREL = 8
_H = 128
_T = 3
_E = 320000
_ES = 4096

_NC = 2
_NS = 16
_L = 16
_GR = 16
_SLOPE = (1.0 / 8.0 + 1.0 / 3.0) / 2.0

_MESH = plsc.VectorSubcoreMesh(core_axis_name="c", subcore_axis_name="s",
                               num_cores=_NC, num_subcores=_NS)


def _chunk_size(ew):
    for cand in range(128, 0, -16):
        if ew % cand == 0:
            return cand
    raise ValueError(ew)


def _chunk_size_db(ew):
    for cand in range(64, 0, -8):
        if ew % cand == 0 and (ew // cand) % 2 == 0:
            return cand
    raise ValueError(ew)


def _build_edge_pass(n_nodes, n_edges):
    ew = n_edges // (_NC * _NS)
    ew_pad = -(-ew // 128) * 128
    if (ew_pad // 64) % 2:
        ew_pad += 64
    c = 64
    chunks = ew_pad // c
    pad = ew_pad - ew
    n_acc = n_nodes + (8 if pad else 0)
    n_groups = n_nodes // _GR
    g_iters = -(-n_groups // _NS)

    @functools.partial(
        pl.kernel,
        out_type=jax.ShapeDtypeStruct((_NC, n_nodes, _H), jnp.float32),
        mesh=_MESH,
        scratch_types=[
            pltpu.VMEM((3 * c,), jnp.int32),
            pltpu.VMEM((3 * c,), jnp.int32),
            pltpu.VMEM((c, _H), jnp.float32),
            pltpu.VMEM((c, _H), jnp.float32),
            pltpu.VMEM((c, _H), jnp.float32),
            pltpu.VMEM((c, _H), jnp.float32),
            pltpu.VMEM_SHARED((n_acc, _H), jnp.float32),
            pltpu.SemaphoreType.DMA,
            pltpu.SemaphoreType.DMA,
            pltpu.SemaphoreType.DMA,
            pltpu.SemaphoreType.DMA,
        ],
    )
    def edge_pass(ntab, rtab, idx_h, z_h, out_acc,
                  idxb0, idxb1, nodeb0, nodeb1, relb0, relb1, acc_sh,
                  isem0, isem1, gsem0, gsem1):
        idxb = (idxb0, idxb1)
        nodeb = (nodeb0, nodeb1)
        relb = (relb0, relb1)
        isem = (isem0, isem1)
        gsem = (gsem0, gsem1)
        cid = lax.axis_index("c")
        sid = lax.axis_index("s")
        wid = cid * _NS + sid

        def zero_group(i, _):
            gid = i * _NS + sid

            @pl.when(gid < n_groups)
            def _():
                pltpu.sync_copy(z_h,
                                acc_sh.at[pl.ds(pl.multiple_of(gid * _GR, 8),
                                                _GR)])

            return 0

        lax.fori_loop(0, g_iters, zero_group, 0)
        plsc.subcore_barrier()

        def idx_slice(g):
            return idx_h.at[pl.ds(
                pl.multiple_of((wid * chunks + g) * (3 * c), 8), 3 * c)]

        def fire_idx(g, b):
            pltpu.async_copy(idx_slice(g), idxb[b], isem[b])

        def wait_idx(b):
            pltpu.make_async_copy(idx_h.at[pl.ds(0, 3 * c)], idxb[b],
                                  isem[b]).wait()

        def fire_gather(b):
            pltpu.async_copy(ntab.at[idxb[b].at[pl.ds(0, c)]], nodeb[b],
                             gsem[b])
            pltpu.async_copy(rtab.at[idxb[b].at[pl.ds(2 * c, c)]], relb[b],
                             gsem[b])

        def wait_gather(b):
            pltpu.make_async_copy(ntab.at[pl.ds(0, c)], nodeb[b],
                                  gsem[b]).wait()
            pltpu.make_async_copy(ntab.at[pl.ds(0, c)], relb[b],
                                  gsem[b]).wait()

        pltpu.sync_copy(idx_slice(0), idxb[0])
        fire_gather(0)
        fire_idx(1, 1)

        def pair_body(gp, _):
            for b in (0, 1):
                g = gp * 2 + b
                nb = 1 - b
                wait_gather(b)

                @pl.when(g + 1 < chunks)
                def _():
                    wait_idx(nb)
                    fire_gather(nb)

                def erow(r, _2):
                    r2 = r * 2
                    for cg in range(_H // _L):
                        sl = pl.ds(cg * _L, _L)
                        nodeb[b][r2, sl] = nodeb[b][r2, sl] * relb[b][r2, sl]
                    for cg in range(_H // _L):
                        sl = pl.ds(cg * _L, _L)
                        nodeb[b][r2 + 1, sl] = (nodeb[b][r2 + 1, sl]
                                                * relb[b][r2 + 1, sl])
                    return 0

                lax.fori_loop(0, c // 2, erow, 0)
                pltpu.sync_copy(nodeb[b], acc_sh.at[idxb[b].at[pl.ds(c, c)]],
                                add=True)

                @pl.when(g + 2 < chunks)
                def _():
                    fire_idx(g + 2, b)

            return 0

        lax.fori_loop(0, chunks // 2, pair_body, 0)
        plsc.subcore_barrier()

        def dump_group(i, _):
            gid = i * _NS + sid

            @pl.when(gid < n_groups)
            def _():
                r0 = pl.multiple_of(gid * _GR, 8)
                pltpu.sync_copy(acc_sh.at[pl.ds(r0, _GR)],
                                out_acc.at[cid, pl.ds(r0, _GR)])

            return 0

        lax.fori_loop(0, g_iters, dump_group, 0)

    def run(ntab, rtab, src, dst, ety):
        nw = _NC * _NS
        if pad:
            padw = lambda a, v: jnp.pad(a.reshape(nw, ew), ((0, 0), (0, pad)),
                                        constant_values=v)
            src = padw(src, 0)
            dst = padw(dst, n_nodes)
            ety = padw(ety, 0)
        idx = jnp.stack([src.reshape(nw, chunks, c),
                         dst.reshape(nw, chunks, c),
                         ety.reshape(nw, chunks, c)], axis=2).reshape(-1)
        z = jnp.zeros((_GR, _H), jnp.float32)
        return edge_pass(ntab, rtab, idx, z)

    return run


def _build_deg_pass(n_nodes, n_edges):
    ew = n_edges // (_NC * _NS)
    c = _chunk_size_db(ew)
    chunks = ew // c
    n_groups = n_nodes // _GR
    g_iters = -(-n_groups // _NS)

    @functools.partial(
        pl.kernel,
        out_type=jax.ShapeDtypeStruct((_T, _NC, n_nodes, _H), jnp.float32),
        mesh=_MESH,
        scratch_types=[
            pltpu.VMEM((c,), jnp.int32),
            pltpu.VMEM((c,), jnp.int32),
            pltpu.VMEM((c, _H), jnp.float32),
            pltpu.VMEM_SHARED((n_nodes, _H), jnp.float32),
            pltpu.SemaphoreType.DMA,
            pltpu.SemaphoreType.DMA,
            pltpu.SemaphoreType.DMA,
            pltpu.SemaphoreType.DMA,
        ],
    )
    def deg_pass(dst0_h, dst1_h, dst2_h, ones_h, z_h, out_deg,
                 dstb0, dstb1, onesb, deg_sh, isem0, isem1, ssem0, ssem1):
        dstb = (dstb0, dstb1)
        isem = (isem0, isem1)
        ssem = (ssem0, ssem1)
        cid = lax.axis_index("c")
        sid = lax.axis_index("s")
        wid = cid * _NS + sid

        pltpu.sync_copy(ones_h, onesb)

        for t, dst_h in enumerate((dst0_h, dst1_h, dst2_h)):
            def zero_group(i, _):
                gid = i * _NS + sid

                @pl.when(gid < n_groups)
                def _():
                    pltpu.sync_copy(
                        z_h,
                        deg_sh.at[pl.ds(pl.multiple_of(gid * _GR, 8), _GR)])

                return 0

            lax.fori_loop(0, g_iters, zero_group, 0)
            plsc.subcore_barrier()

            def fire_idx(g, b):
                pltpu.async_copy(
                    dst_h.at[pl.ds(pl.multiple_of(wid * ew + g * c, 8), c)],
                    dstb[b], isem[b])

            def wait_idx(b):
                pltpu.make_async_copy(dst_h.at[pl.ds(0, c)], dstb[b],
                                      isem[b]).wait()

            def wait_scat(b):
                pltpu.make_async_copy(onesb, deg_sh.at[pl.ds(0, c)],
                                      ssem[b]).wait()

            fire_idx(0, 0)

            def pair_body(gp, _):
                for b in (0, 1):
                    g = gp * 2 + b
                    nb = 1 - b
                    wait_idx(b)
                    pltpu.async_copy(onesb, deg_sh.at[dstb[b]], ssem[b],
                                     add=True)

                    @pl.when(g >= 1)
                    def _():
                        wait_scat(nb)

                    @pl.when(g + 1 < chunks)
                    def _():
                        fire_idx(g + 1, nb)

                return 0

            lax.fori_loop(0, chunks // 2, pair_body, 0)
            wait_scat(1)
            plsc.subcore_barrier()

            def dump_group(i, _):
                gid = i * _NS + sid

                @pl.when(gid < n_groups)
                def _():
                    r0 = pl.multiple_of(gid * _GR, 8)
                    pltpu.sync_copy(deg_sh.at[pl.ds(r0, _GR)],
                                    out_deg.at[t, cid, pl.ds(r0, _GR)])

                return 0

            lax.fori_loop(0, g_iters, dump_group, 0)
            plsc.subcore_barrier()

    def run(dst_all):
        ones = jnp.ones((c, _H), jnp.float32)
        z = jnp.zeros((_GR, _H), jnp.float32)
        return deg_pass(dst_all[0], dst_all[1], dst_all[2], ones, z)

    return run


_EDGE_PASS_SUPER = _build_edge_pass(_NUM_RELS2, _ES)
_EDGE_PASS_ENT = _build_edge_pass(_NUM_ENTS, _E)
_DEG_PASS_SUPER = _build_deg_pass(_NUM_RELS2, _ES)
_DEG_PASS_ENT = _build_deg_pass(_NUM_ENTS, _E)


def _norm_body(x_ref, o_ref):
    x = x_ref[...]
    n = jnp.sqrt(jnp.sum(x * x, axis=1, keepdims=True))
    o_ref[...] = x / jnp.maximum(n, 1e-12)


def _normalize(x, block):
    n = x.shape[0]
    return pl.pallas_call(
        _norm_body,
        out_shape=jax.ShapeDtypeStruct(x.shape, x.dtype),
        grid=(n // block,),
        in_specs=[pl.BlockSpec((block, _H), lambda i: (i, 0))],
        out_specs=pl.BlockSpec((block, _H), lambda i: (i, 0)),
    )(x)


def _update_body(h_ref, acc_ref, dacc_ref, w_ref, ws_ref, gw_ref, gb_ref,
                 o_ref):
    h = h_ref[...]
    acc = acc_ref[0] + acc_ref[1]
    dg = dacc_ref[0] + dacc_ref[1]
    deg = dg[:, 0:1]
    agg = acc / jnp.maximum(deg, 1.0)
    out = (jnp.dot(agg, w_ref[...], preferred_element_type=jnp.float32)
           + jnp.dot(h, ws_ref[...], preferred_element_type=jnp.float32))
    out = jnp.where(out >= 0, out, _SLOPE * out)
    n1 = jnp.sqrt(jnp.sum(out * out, axis=1, keepdims=True))
    cur = out / jnp.maximum(n1, 1e-12)
    g = jax.nn.sigmoid(
        jnp.dot(h, gw_ref[...], preferred_element_type=jnp.float32)
        + gb_ref[...])
    nh = g * cur + (1.0 - g) * h
    n2 = jnp.sqrt(jnp.sum(nh * nh, axis=1, keepdims=True))
    o_ref[...] = nh / jnp.maximum(n2, 1e-12)


def _update(h, acc, dacc, w, w_self, gate_w, gate_b, block):
    n = h.shape[0]
    full = lambda i: (0, 0)
    return pl.pallas_call(
        _update_body,
        out_shape=jax.ShapeDtypeStruct((n, _H), jnp.float32),
        grid=(n // block,),
        in_specs=[
            pl.BlockSpec((block, _H), lambda i: (i, 0)),
            pl.BlockSpec((_NC, block, _H), lambda i: (0, i, 0)),
            pl.BlockSpec((_NC, block, _H), lambda i: (0, i, 0)),
            pl.BlockSpec((_H, _H), full),
            pl.BlockSpec((_H, _H), full),
            pl.BlockSpec((_H, _H), full),
            pl.BlockSpec((1, _H), full),
        ],
        out_specs=pl.BlockSpec((block, _H), lambda i: (i, 0)),
    )(h, acc, dacc, w, w_self, gate_w, gate_b.reshape(1, _H))


def kernel(dynamic_emb, emb_rel, p_rel, rel_gate_w, rel_gate_b, node_gate_w,
           node_gate_b, W_node, W_self_node, W_rel, W_self_rel,
           edge_index, edge_type, super_edge_index, super_edge_type):
    i32 = jnp.int32
    deg_s_all = _DEG_PASS_SUPER(super_edge_index[:, 1].astype(i32))
    deg_n_all = _DEG_PASS_ENT(edge_index[:, 1].astype(i32))
    nodes = _normalize(dynamic_emb, 1000)
    rels = _normalize(emb_rel, _NUM_RELS2)
    history = []
    for t in range(_T):
        acc_s = _EDGE_PASS_SUPER(
            rels, p_rel,
            super_edge_index[t, 0].astype(i32),
            super_edge_index[t, 1].astype(i32),
            super_edge_type[t].astype(i32))
        rels = _update(rels, acc_s, deg_s_all[t], W_rel, W_self_rel,
                       rel_gate_w, rel_gate_b, _NUM_RELS2)
        acc_n = _EDGE_PASS_ENT(
            nodes, rels,
            edge_index[t, 0].astype(i32),
            edge_index[t, 1].astype(i32),
            edge_type[t].astype(i32))
        nodes = _update(nodes, acc_n, deg_n_all[t], W_node, W_self_node,
                        node_gate_w, node_gate_b, 1000)
        history.append(nodes)
    return (jnp.stack(history), rels)


# speedup vs baseline: 1.3265x; 1.3265x over previous
import functools

import jax
import jax.numpy as jnp
from jax import lax
from jax.experimental import pallas as pl
from jax.experimental.pallas import tpu as pltpu
from jax.experimental.pallas import tpu_sc as plsc

_NUM_ENTS = 10000
_NUM_RELS2 = 400
_NUM_PREL = 8
_H = 128
_T = 3
_E = 320000
_ES = 4096

_NC = 2
_NS = 16
_L = 16
_GR = 16
_SLOPE = (1.0 / 8.0 + 1.0 / 3.0) / 2.0

_MESH = plsc.VectorSubcoreMesh(core_axis_name="c", subcore_axis_name="s",
                               num_cores=_NC, num_subcores=_NS)


def _chunk_size(ew):
    for cand in range(128, 0, -16):
        if ew % cand == 0:
            return cand
    raise ValueError(ew)


def _chunk_size_db(ew):
    for cand in range(64, 0, -8):
        if ew % cand == 0 and (ew // cand) % 2 == 0:
            return cand
    raise ValueError(ew)


def _build_edge_pass(n_nodes, n_edges):
    ew = n_edges // (_NC * _NS)
    c = _chunk_size_db(ew)
    chunks = ew // c
    n_groups = n_nodes // _GR
    g_iters = -(-n_groups // _NS)

    @functools.partial(
        pl.kernel,
        out_type=jax.ShapeDtypeStruct((_NC, n_nodes, _H), jnp.float32),
        mesh=_MESH,
        scratch_types=[
            pltpu.VMEM((3 * c,), jnp.int32),
            pltpu.VMEM((3 * c,), jnp.int32),
            pltpu.VMEM((c, _H), jnp.float32),
            pltpu.VMEM((c, _H), jnp.float32),
            pltpu.VMEM((c, _H), jnp.float32),
            pltpu.VMEM((c, _H), jnp.float32),
            pltpu.VMEM((_GR, _H), jnp.float32),
            pltpu.VMEM_SHARED((n_nodes, _H), jnp.float32),
            pltpu.SemaphoreType.DMA,
            pltpu.SemaphoreType.DMA,
            pltpu.SemaphoreType.DMA,
            pltpu.SemaphoreType.DMA,
        ],
    )
    def edge_pass(ntab, rtab, idx_h, z_h, out_acc,
                  idxb0, idxb1, nodeb0, nodeb1, relb0, relb1, zbuf, acc_sh,
                  isem0, isem1, gsem0, gsem1):
        idxb = (idxb0, idxb1)
        nodeb = (nodeb0, nodeb1)
        relb = (relb0, relb1)
        isem = (isem0, isem1)
        gsem = (gsem0, gsem1)
        cid = lax.axis_index("c")
        sid = lax.axis_index("s")
        wid = cid * _NS + sid

        pltpu.sync_copy(z_h, zbuf)

        def zero_group(i, _):
            gid = i * _NS + sid

            @pl.when(gid < n_groups)
            def _():
                pltpu.sync_copy(zbuf,
                                acc_sh.at[pl.ds(pl.multiple_of(gid * _GR, 8),
                                                _GR)])

            return 0

        lax.fori_loop(0, g_iters, zero_group, 0)
        plsc.subcore_barrier()

        def idx_slice(g):
            return idx_h.at[pl.ds(
                pl.multiple_of((wid * chunks + g) * (3 * c), 8), 3 * c)]

        def fire_idx(g, b):
            pltpu.async_copy(idx_slice(g), idxb[b], isem[b])

        def wait_idx(b):
            pltpu.make_async_copy(idx_h.at[pl.ds(0, 3 * c)], idxb[b],
                                  isem[b]).wait()

        def fire_gather(b):
            pltpu.async_copy(ntab.at[idxb[b].at[pl.ds(0, c)]], nodeb[b],
                             gsem[b])
            pltpu.async_copy(rtab.at[idxb[b].at[pl.ds(2 * c, c)]], relb[b],
                             gsem[b])

        def wait_gather(b):
            pltpu.make_async_copy(ntab.at[pl.ds(0, c)], nodeb[b],
                                  gsem[b]).wait()
            pltpu.make_async_copy(ntab.at[pl.ds(0, c)], relb[b],
                                  gsem[b]).wait()

        pltpu.sync_copy(idx_slice(0), idxb[0])
        fire_gather(0)
        fire_idx(1, 1)

        def pair_body(gp, _):
            for b in (0, 1):
                g = gp * 2 + b
                nb = 1 - b
                wait_gather(b)

                @pl.when(g + 1 < chunks)
                def _():
                    wait_idx(nb)
                    fire_gather(nb)

                def erow(r, _2):
                    r2 = r * 2
                    for cg in range(_H // _L):
                        sl = pl.ds(cg * _L, _L)
                        nodeb[b][r2, sl] = nodeb[b][r2, sl] * relb[b][r2, sl]
                    for cg in range(_H // _L):
                        sl = pl.ds(cg * _L, _L)
                        nodeb[b][r2 + 1, sl] = (nodeb[b][r2 + 1, sl]
                                                * relb[b][r2 + 1, sl])
                    return 0

                lax.fori_loop(0, c // 2, erow, 0)
                pltpu.sync_copy(nodeb[b], acc_sh.at[idxb[b].at[pl.ds(c, c)]],
                                add=True)

                @pl.when(g + 2 < chunks)
                def _():
                    fire_idx(g + 2, b)

            return 0

        lax.fori_loop(0, chunks // 2, pair_body, 0)
        plsc.subcore_barrier()

        def dump_group(i, _):
            gid = i * _NS + sid

            @pl.when(gid < n_groups)
            def _():
                r0 = pl.multiple_of(gid * _GR, 8)
                pltpu.sync_copy(acc_sh.at[pl.ds(r0, _GR)],
                                out_acc.at[cid, pl.ds(r0, _GR)])

            return 0

        lax.fori_loop(0, g_iters, dump_group, 0)

    def run(ntab, rtab, src, dst, ety):
        nw = _NC * _NS
        idx = jnp.stack([src.reshape(nw, chunks, c),
                         dst.reshape(nw, chunks, c),
                         ety.reshape(nw, chunks, c)], axis=2).reshape(-1)
        z = jnp.zeros((_GR, _H), jnp.float32)
        return edge_pass(ntab, rtab, idx, z)

    return run


def _build_deg_pass(n_nodes, n_edges):
    ew = n_edges // (_NC * _NS)
    c = _chunk_size_db(ew)
    chunks = ew // c
    n_groups = n_nodes // _GR
    g_iters = -(-n_groups // _NS)

    @functools.partial(
        pl.kernel,
        out_type=jax.ShapeDtypeStruct((_T, _NC, n_nodes, _H), jnp.float32),
        mesh=_MESH,
        scratch_types=[
            pltpu.VMEM((c,), jnp.int32),
            pltpu.VMEM((c,), jnp.int32),
            pltpu.VMEM((c, _H), jnp.float32),
            pltpu.VMEM((_GR, _H), jnp.float32),
            pltpu.VMEM_SHARED((n_nodes, _H), jnp.float32),
            pltpu.SemaphoreType.DMA,
            pltpu.SemaphoreType.DMA,
            pltpu.SemaphoreType.DMA,
            pltpu.SemaphoreType.DMA,
        ],
    )
    def deg_pass(dst0_h, dst1_h, dst2_h, ones_h, z_h, out_deg,
                 dstb0, dstb1, onesb, zbuf, deg_sh,
                 isem0, isem1, ssem0, ssem1):
        dstb = (dstb0, dstb1)
        isem = (isem0, isem1)
        ssem = (ssem0, ssem1)
        cid = lax.axis_index("c")
        sid = lax.axis_index("s")
        wid = cid * _NS + sid

        pltpu.sync_copy(z_h, zbuf)
        pltpu.sync_copy(ones_h, onesb)

        for t, dst_h in enumerate((dst0_h, dst1_h, dst2_h)):
            def zero_group(i, _):
                gid = i * _NS + sid

                @pl.when(gid < n_groups)
                def _():
                    pltpu.sync_copy(
                        zbuf,
                        deg_sh.at[pl.ds(pl.multiple_of(gid * _GR, 8), _GR)])

                return 0

            lax.fori_loop(0, g_iters, zero_group, 0)
            plsc.subcore_barrier()

            def fire_idx(g, b):
                pltpu.async_copy(
                    dst_h.at[pl.ds(pl.multiple_of(wid * ew + g * c, 8), c)],
                    dstb[b], isem[b])

            def wait_idx(b):
                pltpu.make_async_copy(dst_h.at[pl.ds(0, c)], dstb[b],
                                      isem[b]).wait()

            def wait_scat(b):
                pltpu.make_async_copy(onesb, deg_sh.at[pl.ds(0, c)],
                                      ssem[b]).wait()

            fire_idx(0, 0)

            def pair_body(gp, _):
                for b in (0, 1):
                    g = gp * 2 + b
                    nb = 1 - b
                    wait_idx(b)
                    pltpu.async_copy(onesb, deg_sh.at[dstb[b]], ssem[b],
                                     add=True)

                    @pl.when(g >= 1)
                    def _():
                        wait_scat(nb)

                    @pl.when(g + 1 < chunks)
                    def _():
                        fire_idx(g + 1, nb)

                return 0

            lax.fori_loop(0, chunks // 2, pair_body, 0)
            wait_scat(1)
            plsc.subcore_barrier()

            def dump_group(i, _):
                gid = i * _NS + sid

                @pl.when(gid < n_groups)
                def _():
                    r0 = pl.multiple_of(gid * _GR, 8)
                    pltpu.sync_copy(deg_sh.at[pl.ds(r0, _GR)],
                                    out_deg.at[t, cid, pl.ds(r0, _GR)])

                return 0

            lax.fori_loop(0, g_iters, dump_group, 0)
            plsc.subcore_barrier()

    def run(dst_all):
        ones = jnp.ones((c, _H), jnp.float32)
        z = jnp.zeros((_GR, _H), jnp.float32)
        return deg_pass(dst_all[0], dst_all[1], dst_all[2], ones, z)

    return run


_EDGE_PASS_SUPER = _build_edge_pass(_NUM_RELS2, _ES)
_EDGE_PASS_ENT = _build_edge_pass(_NUM_ENTS, _E)
_DEG_PASS_SUPER = _build_deg_pass(_NUM_RELS2, _ES)
_DEG_PASS_ENT = _build_deg_pass(_NUM_ENTS, _E)


def _norm_body(x_ref, o_ref):
    x = x_ref[...]
    n = jnp.sqrt(jnp.sum(x * x, axis=1, keepdims=True))
    o_ref[...] = x / jnp.maximum(n, 1e-12)


def _normalize(x, block):
    n = x.shape[0]
    return pl.pallas_call(
        _norm_body,
        out_shape=jax.ShapeDtypeStruct(x.shape, x.dtype),
        grid=(n // block,),
        in_specs=[pl.BlockSpec((block, _H), lambda i: (i, 0))],
        out_specs=pl.BlockSpec((block, _H), lambda i: (i, 0)),
    )(x)


def _update_body(h_ref, acc_ref, dacc_ref, w_ref, ws_ref, gw_ref, gb_ref,
                 o_ref):
    h = h_ref[...]
    acc = acc_ref[0] + acc_ref[1]
    dg = dacc_ref[0] + dacc_ref[1]
    deg = dg[:, 0:1]
    agg = acc / jnp.maximum(deg, 1.0)
    out = (jnp.dot(agg, w_ref[...], preferred_element_type=jnp.float32)
           + jnp.dot(h, ws_ref[...], preferred_element_type=jnp.float32))
    out = jnp.where(out >= 0, out, _SLOPE * out)
    n1 = jnp.sqrt(jnp.sum(out * out, axis=1, keepdims=True))
    cur = out / jnp.maximum(n1, 1e-12)
    g = jax.nn.sigmoid(
        jnp.dot(h, gw_ref[...], preferred_element_type=jnp.float32)
        + gb_ref[...])
    nh = g * cur + (1.0 - g) * h
    n2 = jnp.sqrt(jnp.sum(nh * nh, axis=1, keepdims=True))
    o_ref[...] = nh / jnp.maximum(n2, 1e-12)


def _update(h, acc, dacc, w, w_self, gate_w, gate_b, block):
    n = h.shape[0]
    full = lambda i: (0, 0)
    return pl.pallas_call(
        _update_body,
        out_shape=jax.ShapeDtypeStruct((n, _H), jnp.float32),
        grid=(n // block,),
        in_specs=[
            pl.BlockSpec((block, _H), lambda i: (i, 0)),
            pl.BlockSpec((_NC, block, _H), lambda i: (0, i, 0)),
            pl.BlockSpec((_NC, block, _H), lambda i: (0, i, 0)),
            pl.BlockSpec((_H, _H), full),
            pl.BlockSpec((_H, _H), full),
            pl.BlockSpec((_H, _H), full),
            pl.BlockSpec((1, _H), full),
        ],
        out_specs=pl.BlockSpec((block, _H), lambda i: (i, 0)),
    )(h, acc, dacc, w, w_self, gate_w, gate_b.reshape(1, _H))


def kernel(dynamic_emb, emb_rel, p_rel, rel_gate_w, rel_gate_b, node_gate_w,
           node_gate_b, W_node, W_self_node, W_rel, W_self_rel,
           edge_index, edge_type, super_edge_index, super_edge_type):
    i32 = jnp.int32
    deg_s_all = _DEG_PASS_SUPER(super_edge_index[:, 1].astype(i32))
    deg_n_all = _DEG_PASS_ENT(edge_index[:, 1].astype(i32))
    nodes = _normalize(dynamic_emb, 1000)
    rels = _normalize(emb_rel, _NUM_RELS2)
    history = []
    for t in range(_T):
        acc_s = _EDGE_PASS_SUPER(
            rels, p_rel,
            super_edge_index[t, 0].astype(i32),
            super_edge_index[t, 1].astype(i32),
            super_edge_type[t].astype(i32))
        rels = _update(rels, acc_s, deg_s_all[t], W_rel, W_self_rel,
                       rel_gate_w, rel_gate_b, _NUM_RELS2)
        acc_n = _EDGE_PASS_ENT(
            nodes, rels,
            edge_index[t, 0].astype(i32),
            edge_index[t, 1].astype(i32),
            edge_type[t].astype(i32))
        nodes = _update(nodes, acc_n, deg_n_all[t], W_node, W_self_node,
                        node_gate_w, node_gate_b, 1000)
        history.append(nodes)
    return (jnp.stack(history), rels)
